# baseline (speedup 1.0000x reference)
import functools

import jax
import jax.numpy as jnp
from jax import lax
from jax.experimental import pallas as pl
from jax.experimental.pallas import tpu as pltpu
from jax.experimental.pallas import tpu_sc as plsc

N = 10000
E = 320000
D = 128
K = 5000
ITERS = 10

NC = 2
NS = 16
NW = NC * NS

NPAD = 10240
HALF = NPAD // NC
ROWS_PT = HALF // NS
NREAL_HALF = 5000
ZPAD_W = HALF - NREAL_HALF
ZPAD0 = NREAL_HALF
ZPAD1 = HALF + NREAL_HALF

ECH = 128
ECHUNKS = 82
EPT = ECH * ECHUNKS
ECAP = EPT * NS

DCHUNKS = 157
EPAD_DEG = NS * DCHUNKS * ECH

KCH = 64
KCHUNKS = 3
KPT = KCH * KCHUNKS
KCAP = KPT * NS

POLL_BOUND = 1024

f32 = jnp.float32
i32 = jnp.int32


@functools.lru_cache(maxsize=None)
def _mesh():
    return plsc.VectorSubcoreMesh(core_axis_name="c", subcore_axis_name="s",
                                  num_cores=NC, num_subcores=NS)


def _wid():
    return lax.axis_index("c") * NS + lax.axis_index("s")


def _zero_vmem(ref, rows):
    @pl.loop(0, rows)
    def _(r):
        for j in range(D // 16):
            ref[r, pl.ds(j * 16, 16)] = jnp.zeros((16,), f32)


def _rsqrt16(d):
    dd = jnp.maximum(d, 1.0)
    y = 1.0 / dd
    for _ in range(26):
        y = y * (1.5 - 0.5 * dd * y * y)
    return jnp.where(d > 0.0, y, 0.0)


@functools.lru_cache(maxsize=None)
def _make_setup():
    @functools.partial(
        pl.kernel,
        out_type=(jax.ShapeDtypeStruct((NPAD,), f32),
                  jax.ShapeDtypeStruct((NC, 16), i32)),
        mesh=_mesh(),
        scratch_types=[
            pltpu.VMEM_SHARED((NPAD,), f32),
            pltpu.VMEM((DCHUNKS, ECH), i32),
            pltpu.VMEM((ECH,), f32),
            pltpu.VMEM((NPAD // NS,), f32),
            pltpu.VMEM((ROWS_PT,), f32),
            pltpu.VMEM((ROWS_PT,), f32),
            pltpu.VMEM((16,), i32),
        ],
    )
    def setup(erow_deg, disH, flagH, degS, didx, ones, zslice, degL, disL,
              fzero):
        c = lax.axis_index("c")
        s = lax.axis_index("s")

        @pl.loop(0, NPAD // NS // 16)
        def _(j):
            zslice[pl.ds(j * 16, 16)] = jnp.zeros((16,), f32)

        pltpu.sync_copy(zslice, degS.at[pl.ds(s * (NPAD // NS), NPAD // NS)])

        @pl.loop(0, ECH // 16)
        def _(j):
            ones[pl.ds(j * 16, 16)] = jnp.full((16,), 1.0, f32)

        @pl.when(s == 0)
        def _():
            fzero[pl.ds(0, 16)] = jnp.zeros((16,), i32)
            pltpu.sync_copy(fzero, flagH.at[c])

        pltpu.sync_copy(erow_deg.at[s], didx)
        plsc.subcore_barrier()
        @pl.loop(0, DCHUNKS)
        def _(ch):
            pltpu.sync_copy(ones, degS.at[didx.at[ch]], add=True)

        plsc.subcore_barrier()
        row0 = c * HALF + s * ROWS_PT
        pltpu.sync_copy(degS.at[pl.ds(row0, ROWS_PT)], degL)

        @pl.loop(0, ROWS_PT // 16)
        def _(j):
            disL[pl.ds(j * 16, 16)] = _rsqrt16(degL[pl.ds(j * 16, 16)])

        pltpu.sync_copy(disL, disH.at[pl.ds(row0, ROWS_PT)])

    return setup


@functools.lru_cache(maxsize=None)
def _make_main():
    @functools.partial(
        pl.kernel,
        out_type=(jax.ShapeDtypeStruct((NPAD, D), f32),
                  jax.ShapeDtypeStruct((NPAD, D), f32),
                  jax.ShapeDtypeStruct((NPAD, D), f32)),
        mesh=_mesh(),
        scratch_types=[
            pltpu.VMEM_SHARED((HALF, D), f32),
            pltpu.VMEM((ECHUNKS, ECH), i32),
            pltpu.VMEM((ECHUNKS, ECH), i32),
            pltpu.VMEM((ECH, D), f32),
            pltpu.VMEM((ECH, D), f32),
            pltpu.VMEM((32, D), f32),
            pltpu.VMEM((KCHUNKS, KCH), i32),
            pltpu.VMEM((KCHUNKS, KCH), i32),
            pltpu.VMEM((KPT, D), f32),
            pltpu.VMEM((KCH + 16,), f32),
            pltpu.VMEM((ROWS_PT + 16,), f32),
            pltpu.VMEM((ROWS_PT + 16,), f32),
            pltpu.VMEM((16,), i32),
            pltpu.VMEM((16,), i32),
            pltpu.SemaphoreType.DMA,
            pltpu.SemaphoreType.DMA,
        ],
    )
    def main(x, erow2d, ecol2d, kidx32, kvx32, disH, flagH,
             out, vA, vB,
             accum, erowL, ecolL, gbuf0, gbuf1, zblock, kidxL, kvxL,
             xkL, dchunk, disL, dis2L, flagL, fpub, sem0, sem1):
        c = lax.axis_index("c")
        s = lax.axis_index("s")
        g = _wid()
        row0 = c * HALF + s * ROWS_PT
        other = 1 - c

        def publish(val):
            @pl.when(s == 0)
            def _():
                fpub[pl.ds(0, 16)] = jnp.full((16,), val, i32)
                pltpu.sync_copy(fpub, flagH.at[c])

        def wait_other(target):
            @pl.loop(0, POLL_BOUND, init_carry=jnp.int32(-1))
            def _(i, val):
                @pl.when(val < target)
                def _():
                    pltpu.sync_copy(flagH.at[other], flagL)

                return jnp.maximum(val, flagL[pl.ds(0, 16)][0])

        pltpu.sync_copy(erow2d.at[g], erowL)
        pltpu.sync_copy(ecol2d.at[g], ecolL)
        pltpu.sync_copy(kidx32.at[g], kidxL)
        pltpu.sync_copy(kvx32.at[g], kvxL)
        pltpu.sync_copy(disH.at[pl.ds(row0, ROWS_PT)],
                        disL.at[pl.ds(0, ROWS_PT)])

        @pl.loop(0, ROWS_PT // 16)
        def _(j):
            dv = disL[pl.ds(j * 16, 16)]
            dis2L[pl.ds(j * 16, 16)] = dv * dv

        _zero_vmem(zblock, 32)
        for b in range(ROWS_PT // 32):
            pltpu.sync_copy(zblock, vA.at[pl.ds(row0 + b * 32, 32)])
            pltpu.sync_copy(zblock,
                            accum.at[pl.ds(s * ROWS_PT + b * 32, 32)])
        for ch in range(KCHUNKS):
            xb = xkL.at[pl.ds(ch * KCH, KCH)]
            pltpu.async_copy(x.at[kvxL.at[ch]], xb, sem0).wait()
            pltpu.async_copy(disH.at[kidxL.at[ch]],
                             dchunk.at[pl.ds(0, KCH)], sem1).wait()

            @pl.loop(0, KCH)
            def _(r):
                sc = dchunk[pl.ds(r, 16)][0]
                for j in range(D // 16):
                    xkL[ch * KCH + r, pl.ds(j * 16, 16)] = (
                        xkL[ch * KCH + r, pl.ds(j * 16, 16)] * sc)

        plsc.subcore_barrier()
        for ch in range(KCHUNKS):
            pltpu.sync_copy(xkL.at[pl.ds(ch * KCH, KCH)],
                            vA.at[kidxL.at[ch]])
        plsc.subcore_barrier()
        publish(1)

        for t in range(ITERS):
            last = t == ITERS - 1
            src = vA if t % 2 == 0 else vB
            dst = out if last else (vB if t % 2 == 0 else vA)
            wait_other(t + 1)
            pltpu.async_copy(src.at[ecolL.at[0]], gbuf0, sem0)

            @pl.loop(0, ECHUNKS, step=2)
            def _(ch):
                pltpu.async_copy(src.at[ecolL.at[ch + 1]], gbuf1, sem1)
                pltpu.make_async_copy(src.at[ecolL.at[ch]], gbuf0,
                                      sem0).wait()
                pltpu.sync_copy(gbuf0, accum.at[erowL.at[ch]], add=True)

                @pl.when(ch + 2 < ECHUNKS)
                def _():
                    pltpu.async_copy(src.at[ecolL.at[ch + 2]], gbuf0, sem0)

                pltpu.make_async_copy(src.at[ecolL.at[ch + 1]], gbuf1,
                                      sem1).wait()
                pltpu.sync_copy(gbuf1, accum.at[erowL.at[ch + 1]], add=True)

            plsc.subcore_barrier()
            scaleL = disL if last else dis2L
            for b in range(ROWS_PT // KCH):
                ob = gbuf0.at[pl.ds(0, KCH)]
                a0 = s * ROWS_PT + b * KCH
                pltpu.sync_copy(accum.at[pl.ds(a0, KCH)], ob)
                pltpu.sync_copy(zblock, accum.at[pl.ds(a0, 32)])
                pltpu.sync_copy(zblock, accum.at[pl.ds(a0 + 32, 32)])

                @pl.loop(0, KCH)
                def _(r):
                    sc = scaleL[pl.ds(b * KCH + r, 16)][0]
                    for j in range(D // 16):
                        gbuf0[r, pl.ds(j * 16, 16)] = (
                            gbuf0[r, pl.ds(j * 16, 16)] * sc)

                pltpu.sync_copy(ob, dst.at[pl.ds(row0 + b * KCH, KCH)])
            plsc.subcore_barrier()
            for ch in range(KCHUNKS):
                if last:
                    kb = gbuf1.at[pl.ds(0, KCH)]
                    pltpu.async_copy(x.at[kvxL.at[ch]], kb, sem1).wait()
                    pltpu.sync_copy(kb, dst.at[kidxL.at[ch]])
                else:
                    pltpu.sync_copy(xkL.at[pl.ds(ch * KCH, KCH)],
                                    dst.at[kidxL.at[ch]])
            if not last:
                plsc.subcore_barrier()
                publish(t + 2)

    return main


def _buf(r):
    return jnp.where(r < NREAL_HALF, r, r + ZPAD_W)


def _prep_inputs(x, edge_index, known_feature_mask):
    row = edge_index[0].astype(i32)
    col = edge_index[1].astype(i32)
    known = known_feature_mask.astype(i32)
    brow = _buf(row)
    bcol = _buf(col)
    bknown = _buf(known)

    npad_deg = EPAD_DEG - E
    pad_rows = ZPAD0 + (jnp.arange(npad_deg, dtype=i32) % ZPAD_W)
    erow_deg = jnp.concatenate([brow, pad_rows]).reshape(NS, DCHUNKS, ECH)

    side = (row >= NREAL_HALF).astype(i32)
    order = jnp.argsort(side, stable=True)
    n0 = E - jnp.sum(side)
    pos = jnp.arange(E, dtype=i32)
    pos_in_half = jnp.where(pos < n0, pos, pos - n0)
    half_id = (pos >= n0).astype(i32)
    lrow_s = (brow - side * HALF)[order]
    col_s = bcol[order]
    slot = jnp.arange(ECAP, dtype=i32)
    pad_lrow = slot % HALF
    pad_col = ZPAD0 + (slot % ZPAD_W)
    erow_cap = jnp.broadcast_to(pad_lrow, (NC, ECAP))
    ecol_cap = jnp.broadcast_to(pad_col, (NC, ECAP))
    erow_cap = erow_cap.at[half_id, pos_in_half].set(lrow_s)
    ecol_cap = ecol_cap.at[half_id, pos_in_half].set(col_s)
    erow2d = erow_cap.reshape(NW, ECHUNKS, ECH)
    ecol2d = ecol_cap.reshape(NW, ECHUNKS, ECH)

    kside = (known >= NREAL_HALF).astype(i32)
    korder = jnp.argsort(kside, stable=True)
    kn0 = K - jnp.sum(kside)
    kpos = jnp.arange(K, dtype=i32)
    kpos_in_half = jnp.where(kpos < kn0, kpos, kpos - kn0)
    khalf_id = (kpos >= kn0).astype(i32)
    kslot = jnp.arange(KCAP, dtype=i32)
    kpad = ZPAD1 + (kslot % ZPAD_W)
    kidx_cap = jnp.broadcast_to(kpad, (NC, KCAP))
    kidx_cap = kidx_cap.at[khalf_id, kpos_in_half].set(bknown[korder])
    kvx_cap = jnp.zeros((NC, KCAP), i32)
    kvx_cap = kvx_cap.at[khalf_id, kpos_in_half].set(known[korder])
    kidx32 = kidx_cap.reshape(NW, KCHUNKS, KCH)
    kvx32 = kvx_cap.reshape(NW, KCHUNKS, KCH)
    return erow_deg, erow2d, ecol2d, kidx32, kvx32


def kernel(x, edge_index, known_feature_mask):
    erow_deg, erow2d, ecol2d, kidx32, kvx32 = _prep_inputs(
        x, edge_index, known_feature_mask)
    disH, flagH = _make_setup()(erow_deg)
    out, _, _ = _make_main()(x, erow2d, ecol2d, kidx32, kvx32, disH, flagH)
    return jnp.concatenate([out[:NREAL_HALF], out[HALF:HALF + NREAL_HALF]])
